# Initial kernel scaffold; baseline (speedup 1.0000x reference)
import jax
import jax.numpy as jnp
from jax.experimental import pallas as pl


def kernel(user_code, item_code, user_emb, item_emb):
    raise NotImplementedError("write your pallas kernel here")


# speedup vs baseline: 1.1687x; 1.1687x over previous
import functools

import jax
import jax.numpy as jnp
from jax import lax
from jax.experimental import pallas as pl
from jax.experimental.pallas import tpu as pltpu
from jax.experimental.pallas import tpu_sc as plsc


def _bcast_lane(vec, j):
    idx = jnp.full((16, 1), j, jnp.int32)
    dn = lax.GatherDimensionNumbers(
        offset_dims=(), collapsed_slice_dims=(0,), start_index_map=(0,))
    return lax.gather(vec, idx, dn, slice_sizes=(1,),
                      mode=lax.GatherScatterMode.PROMISE_IN_BOUNDS)


def _build_sc_kernel(B, L, D, n_workers, users_per_chunk):
    CU = users_per_chunk
    CI = CU * L
    BPW = B // n_workers
    NCHUNK = BPW // CU
    groups = []
    off = 0
    while off < CI:
        sz = min(128, CI - off)
        groups.append((off, sz))
        off += sz

    mesh = plsc.VectorSubcoreMesh(core_axis_name="c", subcore_axis_name="s")
    NC = mesh.num_cores

    @functools.partial(
        pl.kernel,
        out_type=jax.ShapeDtypeStruct((B * L,), jnp.float32),
        mesh=mesh,
        compiler_params=pltpu.CompilerParams(
            needs_layout_passes=False, use_tc_tiling_on_sc=False),
        scratch_types=[
            pltpu.VMEM((CU,), jnp.int32),
            pltpu.VMEM((CI,), jnp.int32),
            pltpu.VMEM((CU, D), jnp.float32),
            pltpu.VMEM((CI + 16, D), jnp.float32),
            pltpu.VMEM((CI + 16,), jnp.float32),
            pltpu.SemaphoreType.DMA,
        ],
    )
    def sc_kernel(ucode_hbm, icode_hbm, uemb_hbm, iemb_hbm, out_hbm,
                  ucode_v, icode_v, urows_v, irows_v, out_v, sem):
        wid = lax.axis_index("s") * NC + lax.axis_index("c")
        iota = lax.iota(jnp.int32, 16)

        def chunk_body(c, _):
            ubase = wid * BPW + c * CU
            ibase = ubase * L
            pltpu.sync_copy(ucode_hbm.at[pl.ds(ubase, CU)], ucode_v)
            pltpu.sync_copy(icode_hbm.at[pl.ds(ibase, CI)], icode_v)
            copies = [pltpu.async_copy(uemb_hbm.at[ucode_v], urows_v, sem)]
            for goff, gsz in groups:
                copies.append(pltpu.async_copy(
                    iemb_hbm.at[icode_v.at[pl.ds(goff, gsz)]],
                    irows_v.at[pl.ds(goff, gsz)], sem))
            for cp in copies:
                cp.wait()

            def user_body(u, _):
                rbase = u * L
                accs = [jnp.zeros((16,), jnp.float32) for _ in range(4)]
                ridx = [rbase + ci * 16 + iota for ci in range(4)]
                uhalf = [urows_v[u, pl.ds(h * 16, 16)] for h in range(D // 16)]
                for dd in range(D):
                    uvec = _bcast_lane(uhalf[dd // 16], dd % 16)
                    cidx = jnp.full((16,), dd, jnp.int32)
                    for ci in range(4):
                        vals = plsc.load_gather(irows_v, [ridx[ci], cidx])
                        accs[ci] = accs[ci] + vals * uvec
                for ci in range(4):
                    out_v[pl.ds(rbase + ci * 16, 16)] = accs[ci]
                return _

            lax.fori_loop(0, CU, user_body, 0, unroll=False)
            pltpu.sync_copy(out_v.at[pl.ds(0, CI)], out_hbm.at[pl.ds(ibase, CI)])
            return _

        lax.fori_loop(0, NCHUNK, chunk_body, 0, unroll=False)

    return sc_kernel


def kernel(user_code, item_code, user_emb, item_emb):
    B, L = item_code.shape
    D = user_emb.shape[1]
    sck = _build_sc_kernel(B, L, D, n_workers=32, users_per_chunk=32)
    out_flat = sck(user_code, item_code.reshape(-1), user_emb, item_emb)
    return out_flat.reshape(B, L)


# speedup vs baseline: 1.7516x; 1.4988x over previous
import functools

import jax
import jax.numpy as jnp
from jax import lax
from jax.experimental import pallas as pl
from jax.experimental.pallas import tpu as pltpu
from jax.experimental.pallas import tpu_sc as plsc

def _rotate_lanes(vec, idx):
    dn = lax.GatherDimensionNumbers(
        offset_dims=(), collapsed_slice_dims=(0,), start_index_map=(0,))
    return lax.gather(vec, idx[:, None], dn, slice_sizes=(1,),
                      mode=lax.GatherScatterMode.PROMISE_IN_BOUNDS)


def _build_sc_kernel(B, L, D, n_workers, users_per_chunk):
    CU = users_per_chunk
    CI = CU * L
    BPW = B // n_workers
    NCHUNK = BPW // CU
    assert NCHUNK % 2 == 0
    groups = []
    off = 0
    while off < CI:
        sz = min(128, CI - off)
        groups.append((off, sz))
        off += sz

    mesh = plsc.VectorSubcoreMesh(core_axis_name="c", subcore_axis_name="s")
    NC = mesh.num_cores

    @functools.partial(
        pl.kernel,
        out_type=jax.ShapeDtypeStruct((B * L,), jnp.float32),
        mesh=mesh,
        compiler_params=pltpu.CompilerParams(
            needs_layout_passes=False, use_tc_tiling_on_sc=False),
        scratch_types=[
            pltpu.VMEM((2, CU), jnp.int32),
            pltpu.VMEM((2, CI), jnp.int32),
            pltpu.VMEM((2, CU, D), jnp.float32),
            pltpu.VMEM((2, CI + 16, D), jnp.float32),
            pltpu.VMEM((2, CI + 16), jnp.float32),
            pltpu.SemaphoreType.DMA,
            pltpu.SemaphoreType.DMA,
            pltpu.SemaphoreType.DMA,
            pltpu.SemaphoreType.DMA,
        ],
    )
    def sc_kernel(ucode_hbm, icode_hbm, uemb_hbm, iemb_hbm, out_hbm,
                  ucode_v, icode_v, urows_v, irows_v, out_v,
                  sem0, sem1, osem0, osem1):
        wid = lax.axis_index("s") * NC + lax.axis_index("c")
        iota = lax.iota(jnp.int32, 16)
        sems = [sem0, sem1]
        osems = [osem0, osem1]

        def in_copies(c, slot):
            ubase = wid * BPW + c * CU
            ibase = ubase * L
            cps = [pltpu.make_async_copy(
                uemb_hbm.at[ucode_v.at[slot]], urows_v.at[slot], sems[slot])]
            for goff, gsz in groups:
                cps.append(pltpu.make_async_copy(
                    iemb_hbm.at[icode_v.at[slot, pl.ds(goff, gsz)]],
                    irows_v.at[slot, pl.ds(goff, gsz)],
                    sems[slot]))
            return cps

        def out_copy(c, slot):
            ibase = (wid * BPW + c * CU) * L
            return pltpu.make_async_copy(
                out_v.at[slot, pl.ds(0, CI)],
                out_hbm.at[pl.ds(ibase, CI)], osems[slot])

        def issue(c, slot):
            ubase = wid * BPW + c * CU
            ibase = ubase * L
            pltpu.sync_copy(ucode_hbm.at[pl.ds(ubase, CU)],
                            ucode_v.at[slot])
            pltpu.sync_copy(icode_hbm.at[pl.ds(ibase, CI)],
                            icode_v.at[slot])
            for cp in in_copies(c, slot):
                cp.start()

        def compute(c, slot):
            @pl.when(c >= 2)
            def _():
                out_copy(c - 2, slot).wait()

            def user_body(u, carry):
                rbase = u * L
                accs = [jnp.zeros((16,), jnp.float32) for _ in range(4)]
                ridx = [rbase + ci * 16 + iota for ci in range(4)]
                uhalf = [urows_v[slot, u, pl.ds(h * 16, 16)]
                         for h in range(D // 16)]
                for hh in range(D // 16):
                    for s in range(16):
                        rot = (iota + s) & 15
                        urot = _rotate_lanes(uhalf[hh], rot)
                        diag = rot + 16 * hh if hh else rot
                        for ci in range(4):
                            vals = plsc.load_gather(irows_v.at[slot],
                                                    [ridx[ci], diag])
                            accs[ci] = accs[ci] + vals * urot
                for ci in range(4):
                    out_v[slot, pl.ds(rbase + ci * 16, 16)] = accs[ci]
                return carry

            lax.fori_loop(0, CU, user_body, 0, unroll=False)
            out_copy(c, slot).start()

        def wait_in(c, slot):
            for cp in in_copies(c, slot):
                cp.wait()

        issue(0, 0)

        def pair_body(k, carry):
            c0 = 2 * k
            issue(c0 + 1, 1)
            wait_in(c0, 0)
            compute(c0, 0)

            @pl.when(k < NCHUNK // 2 - 1)
            def _():
                issue(c0 + 2, 0)

            wait_in(c0 + 1, 1)
            compute(c0 + 1, 1)
            return carry

        lax.fori_loop(0, NCHUNK // 2, pair_body, 0, unroll=False)
        out_copy(NCHUNK - 2, 0).wait()
        out_copy(NCHUNK - 1, 1).wait()

    return sc_kernel


def kernel(user_code, item_code, user_emb, item_emb):
    B, L = item_code.shape
    D = user_emb.shape[1]
    sck = _build_sc_kernel(B, L, D, n_workers=32, users_per_chunk=32)
    out_flat = sck(user_code, item_code.reshape(-1), user_emb, item_emb)
    return out_flat.reshape(B, L)


# speedup vs baseline: 2.8093x; 1.6038x over previous
import functools

import jax
import jax.numpy as jnp
from jax import lax
from jax.experimental import pallas as pl
from jax.experimental.pallas import tpu as pltpu
from jax.experimental.pallas import tpu_sc as plsc


def _rotate_lanes(vec, idx):
    dn = lax.GatherDimensionNumbers(
        offset_dims=(), collapsed_slice_dims=(0,), start_index_map=(0,))
    return lax.gather(vec, idx[:, None], dn, slice_sizes=(1,),
                      mode=lax.GatherScatterMode.PROMISE_IN_BOUNDS)


def _build_relayout_kernel(V, D, n_workers):
    KD = D // 8
    NKR = (V + 127) // 128
    VP = NKR * 128
    PERW = (NKR + n_workers - 1) // n_workers
    mesh = plsc.VectorSubcoreMesh(core_axis_name="c", subcore_axis_name="s")
    NC = mesh.num_cores

    @functools.partial(
        pl.kernel,
        out_type=[jax.ShapeDtypeStruct((VP * D,), jnp.float32),
                  jax.ShapeDtypeStruct((VP * D,), jnp.float32)],
        mesh=mesh,
        compiler_params=pltpu.CompilerParams(
            needs_layout_passes=False, use_tc_tiling_on_sc=True,
            disable_bounds_checks=True),
        scratch_types=[
            pltpu.VMEM((D, 128), jnp.float32),
            pltpu.VMEM((D, 128), jnp.float32),
            pltpu.VMEM((128 * D,), jnp.float32),
            pltpu.VMEM((128 * D,), jnp.float32),
            pltpu.SemaphoreType.DMA,
            pltpu.SemaphoreType.DMA,
            pltpu.SemaphoreType.DMA,
            pltpu.SemaphoreType.DMA,
        ],
    )
    def relayout(uembt_hbm, iembt_hbm, uout_hbm, iout_hbm,
                 stage0_v, stage1_v, rows0_v, rows1_v,
                 sem0, sem1, osem0, osem1):
        wid = lax.axis_index("s") * NC + lax.axis_index("c")
        iota = lax.iota(jnp.int32, 16)
        stages = [stage0_v, stage1_v]
        rows = [rows0_v, rows1_v]
        sems = [sem0, sem1]
        osems = [osem0, osem1]

        for src_hbm, out_hbm in ((uembt_hbm, uout_hbm), (iembt_hbm, iout_hbm)):
            def stage_copies(kc, slot, src_hbm=src_hbm):
                return [pltpu.make_async_copy(
                    src_hbm.at[pl.ds(kd * 8, 8), pl.ds(kc * 128, 128)],
                    stages[slot].at[pl.ds(kd * 8, 8)], sems[slot])
                        for kd in range(KD)]

            def out_copy(kc, slot, out_hbm=out_hbm):
                return pltpu.make_async_copy(
                    rows[slot],
                    out_hbm.at[pl.ds(kc * 128 * D, 128 * D)], osems[slot])

            def transpose(kc, slot):
                def rr_body(rr, carry):
                    r16 = rr * 16 + iota
                    r16d = r16 * D
                    for hh in range(D // 16):
                        for s in range(16):
                            dvec = ((iota + s) & 15) + 16 * hh if hh \
                                else (iota + s) & 15
                            vals = plsc.load_gather(stages[slot],
                                                    [dvec, r16])
                            plsc.store_scatter(rows[slot],
                                               [r16d + dvec], vals)
                    return carry

                lax.fori_loop(0, 8, rr_body, 0, unroll=False)
                out_copy(kc, slot).start()

            def issue(kc, slot):
                for cp in stage_copies(kc, slot):
                    cp.start()

            def wait_in(kc, slot):
                for cp in stage_copies(kc, slot):
                    cp.wait()

            nk_pairs = (PERW + 1) // 2
            clamp = lambda kc: jnp.minimum(kc, NKR - 1)
            issue(clamp(wid), 0)

            def pair_body(k, carry):
                kc0 = wid + 2 * k * n_workers
                issue(clamp(kc0 + n_workers), 1)
                wait_in(clamp(kc0), 0)

                @pl.when(k >= 1)
                def _():
                    out_copy(0, 0).wait()

                transpose(clamp(kc0), 0)

                @pl.when(k < nk_pairs - 1)
                def _():
                    issue(clamp(kc0 + 2 * n_workers), 0)

                wait_in(clamp(kc0 + n_workers), 1)

                @pl.when(k >= 1)
                def _():
                    out_copy(0, 1).wait()

                transpose(clamp(kc0 + n_workers), 1)
                return carry

            lax.fori_loop(0, nk_pairs, pair_body, 0, unroll=False)
            out_copy(0, 0).wait()
            out_copy(0, 1).wait()

    return relayout


def _build_main_kernel(B, L, D, n_workers, users_per_chunk):
    CU = users_per_chunk
    CI = CU * L
    BPW = B // n_workers
    NCHUNK = BPW // CU
    assert NCHUNK % 2 == 0
    groups = []
    off = 0
    while off < CI:
        sz = min(128, CI - off)
        groups.append((off, sz))
        off += sz

    mesh = plsc.VectorSubcoreMesh(core_axis_name="c", subcore_axis_name="s")
    NC = mesh.num_cores

    @functools.partial(
        pl.kernel,
        out_type=jax.ShapeDtypeStruct((B * L,), jnp.float32),
        mesh=mesh,
        compiler_params=pltpu.CompilerParams(
            needs_layout_passes=False, use_tc_tiling_on_sc=False),
        scratch_types=[
            pltpu.VMEM((2, CU), jnp.int32),
            pltpu.VMEM((2, CI), jnp.int32),
            pltpu.VMEM((2, CU, D), jnp.float32),
            pltpu.VMEM((2, CI + 16, D), jnp.float32),
            pltpu.VMEM((2, CI + 16), jnp.float32),
            pltpu.SemaphoreType.DMA,
            pltpu.SemaphoreType.DMA,
            pltpu.SemaphoreType.DMA,
            pltpu.SemaphoreType.DMA,
        ],
    )
    def sc_kernel(ucode_hbm, icode_hbm, uemb_hbm, iemb_hbm, out_hbm,
                  ucode_v, icode_v, urows_v, irows_v, out_v,
                  sem0, sem1, osem0, osem1):
        wid = lax.axis_index("s") * NC + lax.axis_index("c")
        iota = lax.iota(jnp.int32, 16)
        sems = [sem0, sem1]
        osems = [osem0, osem1]

        def in_copies(c, slot):
            cps = [pltpu.make_async_copy(
                uemb_hbm.at[ucode_v.at[slot]], urows_v.at[slot], sems[slot])]
            for goff, gsz in groups:
                cps.append(pltpu.make_async_copy(
                    iemb_hbm.at[icode_v.at[slot, pl.ds(goff, gsz)]],
                    irows_v.at[slot, pl.ds(goff, gsz)],
                    sems[slot]))
            return cps

        def out_copy(c, slot):
            ibase = (wid * BPW + c * CU) * L
            return pltpu.make_async_copy(
                out_v.at[slot, pl.ds(0, CI)],
                out_hbm.at[pl.ds(ibase, CI)], osems[slot])

        def issue(c, slot):
            ubase = wid * BPW + c * CU
            ibase = ubase * L
            pltpu.sync_copy(ucode_hbm.at[pl.ds(ubase, CU)],
                            ucode_v.at[slot])
            pltpu.sync_copy(icode_hbm.at[pl.ds(ibase, CI)],
                            icode_v.at[slot])
            for cp in in_copies(c, slot):
                cp.start()

        def compute(c, slot):
            @pl.when(c >= 2)
            def _():
                out_copy(c - 2, slot).wait()

            def user_body(u, carry):
                rbase = u * L
                accs = [jnp.zeros((16,), jnp.float32) for _ in range(4)]
                ridx = [rbase + ci * 16 + iota for ci in range(4)]
                uhalf = [urows_v[slot, u, pl.ds(h * 16, 16)]
                         for h in range(D // 16)]
                for hh in range(D // 16):
                    for s in range(16):
                        rot = (iota + s) & 15
                        urot = _rotate_lanes(uhalf[hh], rot)
                        diag = rot + 16 * hh if hh else rot
                        for ci in range(4):
                            vals = plsc.load_gather(irows_v.at[slot],
                                                    [ridx[ci], diag])
                            accs[ci] = accs[ci] + vals * urot
                for ci in range(4):
                    out_v[slot, pl.ds(rbase + ci * 16, 16)] = accs[ci]
                return carry

            lax.fori_loop(0, CU, user_body, 0, unroll=False)
            out_copy(c, slot).start()

        def wait_in(c, slot):
            for cp in in_copies(c, slot):
                cp.wait()

        issue(0, 0)

        def pair_body(k, carry):
            c0 = 2 * k
            issue(c0 + 1, 1)
            wait_in(c0, 0)
            compute(c0, 0)

            @pl.when(k < NCHUNK // 2 - 1)
            def _():
                issue(c0 + 2, 0)

            wait_in(c0 + 1, 1)
            compute(c0 + 1, 1)
            return carry

        lax.fori_loop(0, NCHUNK // 2, pair_body, 0, unroll=False)
        out_copy(NCHUNK - 2, 0).wait()
        out_copy(NCHUNK - 1, 1).wait()

    return sc_kernel


def kernel(user_code, item_code, user_emb, item_emb):
    B, L = item_code.shape
    V, D = user_emb.shape
    relayout = _build_relayout_kernel(V, D, n_workers=32)
    uflat, iflat = relayout(user_emb.T, item_emb.T)
    vp = uflat.shape[0] // D
    sck = _build_main_kernel(B, L, D, n_workers=32, users_per_chunk=32)
    out_flat = sck(user_code, item_code.reshape(-1),
                   uflat.reshape(vp, D), iflat.reshape(vp, D))
    return out_flat.reshape(B, L)


# speedup vs baseline: 3.4234x; 1.2186x over previous
import functools

import jax
import jax.numpy as jnp
from jax import lax
from jax.experimental import pallas as pl
from jax.experimental.pallas import tpu as pltpu
from jax.experimental.pallas import tpu_sc as plsc


def _rotate_lanes(vec, idx):
    dn = lax.GatherDimensionNumbers(
        offset_dims=(), collapsed_slice_dims=(0,), start_index_map=(0,))
    return lax.gather(vec, idx[:, None], dn, slice_sizes=(1,),
                      mode=lax.GatherScatterMode.PROMISE_IN_BOUNDS)


def _build_relayout_kernel(V, D, n_workers):
    KD = D // 8
    NKR = (V + 127) // 128
    VP = NKR * 128
    PERW = (NKR + n_workers - 1) // n_workers
    mesh = plsc.VectorSubcoreMesh(core_axis_name="c", subcore_axis_name="s")
    NC = mesh.num_cores

    @functools.partial(
        pl.kernel,
        out_type=[jax.ShapeDtypeStruct((VP * D,), jnp.float32),
                  jax.ShapeDtypeStruct((VP * D,), jnp.float32)],
        mesh=mesh,
        compiler_params=pltpu.CompilerParams(
            needs_layout_passes=False, use_tc_tiling_on_sc=True,
            disable_bounds_checks=True),
        scratch_types=[
            pltpu.VMEM((D, 128), jnp.float32),
            pltpu.VMEM((D, 128), jnp.float32),
            pltpu.VMEM((128 * D,), jnp.float32),
            pltpu.VMEM((128 * D,), jnp.float32),
            pltpu.SemaphoreType.DMA,
            pltpu.SemaphoreType.DMA,
            pltpu.SemaphoreType.DMA,
            pltpu.SemaphoreType.DMA,
        ],
    )
    def relayout(uembt_hbm, iembt_hbm, uout_hbm, iout_hbm,
                 stage0_v, stage1_v, rows0_v, rows1_v,
                 sem0, sem1, osem0, osem1):
        wid = lax.axis_index("s") * NC + lax.axis_index("c")
        iota = lax.iota(jnp.int32, 16)
        stages = [stage0_v, stage1_v]
        rows = [rows0_v, rows1_v]
        sems = [sem0, sem1]
        osems = [osem0, osem1]
        rots = [(iota + s) & 15 for s in range(16)]
        rots_hi = [r + 16 for r in rots]

        for src_hbm, out_hbm in ((uembt_hbm, uout_hbm), (iembt_hbm, iout_hbm)):
            def stage_copies(kc, slot, src_hbm=src_hbm):
                return [pltpu.make_async_copy(
                    src_hbm.at[pl.ds(kd * 8, 8), pl.ds(kc * 128, 128)],
                    stages[slot].at[pl.ds(kd * 8, 8)], sems[slot])
                        for kd in range(KD)]

            def out_copy(kc, slot, out_hbm=out_hbm):
                return pltpu.make_async_copy(
                    rows[slot],
                    out_hbm.at[pl.ds(kc * 128 * D, 128 * D)], osems[slot])

            def transpose(kc, slot):
                @plsc.parallel_loop(0, 8, unroll=2)
                def rr_body(rr):
                    r16 = rr * 16 + iota
                    r16d = r16 * D
                    for s in range(16):
                        for dvec in (rots[s], rots_hi[s]):
                            vals = plsc.load_gather(stages[slot],
                                                    [dvec, r16])
                            plsc.store_scatter(rows[slot],
                                               [r16d + dvec], vals)

                out_copy(kc, slot).start()

            def issue(kc, slot):
                for cp in stage_copies(kc, slot):
                    cp.start()

            def wait_in(kc, slot):
                for cp in stage_copies(kc, slot):
                    cp.wait()

            nk_pairs = (PERW + 1) // 2
            clamp = lambda kc: jnp.minimum(kc, NKR - 1)
            issue(clamp(wid), 0)

            def pair_body(k, carry):
                kc0 = wid + 2 * k * n_workers
                issue(clamp(kc0 + n_workers), 1)
                wait_in(clamp(kc0), 0)

                @pl.when(k >= 1)
                def _():
                    out_copy(0, 0).wait()

                transpose(clamp(kc0), 0)

                @pl.when(k < nk_pairs - 1)
                def _():
                    issue(clamp(kc0 + 2 * n_workers), 0)

                wait_in(clamp(kc0 + n_workers), 1)

                @pl.when(k >= 1)
                def _():
                    out_copy(0, 1).wait()

                transpose(clamp(kc0 + n_workers), 1)
                return carry

            lax.fori_loop(0, nk_pairs, pair_body, 0, unroll=False)
            out_copy(0, 0).wait()
            out_copy(0, 1).wait()

    return relayout


def _build_main_kernel(B, L, D, n_workers, users_per_chunk):
    CU = users_per_chunk
    CI = CU * L
    BPW = B // n_workers
    NCHUNK = BPW // CU
    assert NCHUNK % 2 == 0
    groups = []
    off = 0
    while off < CI:
        sz = min(128, CI - off)
        groups.append((off, sz))
        off += sz

    mesh = plsc.VectorSubcoreMesh(core_axis_name="c", subcore_axis_name="s")
    NC = mesh.num_cores

    @functools.partial(
        pl.kernel,
        out_type=jax.ShapeDtypeStruct((B * L,), jnp.float32),
        mesh=mesh,
        compiler_params=pltpu.CompilerParams(
            needs_layout_passes=False, use_tc_tiling_on_sc=False),
        scratch_types=[
            pltpu.VMEM((2, CU), jnp.int32),
            pltpu.VMEM((2, CI), jnp.int32),
            pltpu.VMEM((2, CU, D), jnp.float32),
            pltpu.VMEM((2, CI + 16, D), jnp.float32),
            pltpu.VMEM((2, CI + 16), jnp.float32),
            pltpu.SemaphoreType.DMA,
            pltpu.SemaphoreType.DMA,
            pltpu.SemaphoreType.DMA,
            pltpu.SemaphoreType.DMA,
        ],
    )
    def sc_kernel(ucode_hbm, icode_hbm, uemb_hbm, iemb_hbm, out_hbm,
                  ucode_v, icode_v, urows_v, irows_v, out_v,
                  sem0, sem1, osem0, osem1):
        wid = lax.axis_index("s") * NC + lax.axis_index("c")
        iota = lax.iota(jnp.int32, 16)
        sems = [sem0, sem1]
        osems = [osem0, osem1]

        def in_copies(c, slot):
            cps = [pltpu.make_async_copy(
                uemb_hbm.at[ucode_v.at[slot]], urows_v.at[slot], sems[slot])]
            for goff, gsz in groups:
                cps.append(pltpu.make_async_copy(
                    iemb_hbm.at[icode_v.at[slot, pl.ds(goff, gsz)]],
                    irows_v.at[slot, pl.ds(goff, gsz)],
                    sems[slot]))
            return cps

        def out_copy(c, slot):
            ibase = (wid * BPW + c * CU) * L
            return pltpu.make_async_copy(
                out_v.at[slot, pl.ds(0, CI)],
                out_hbm.at[pl.ds(ibase, CI)], osems[slot])

        def issue(c, slot):
            ubase = wid * BPW + c * CU
            ibase = ubase * L
            pltpu.sync_copy(ucode_hbm.at[pl.ds(ubase, CU)],
                            ucode_v.at[slot])
            pltpu.sync_copy(icode_hbm.at[pl.ds(ibase, CI)],
                            icode_v.at[slot])
            for cp in in_copies(c, slot):
                cp.start()

        def compute(c, slot):
            @pl.when(c >= 2)
            def _():
                out_copy(c - 2, slot).wait()

            def user_body(u, carry):
                rbase = u * L
                accs = [jnp.zeros((16,), jnp.float32) for _ in range(4)]
                ridx = [rbase + ci * 16 + iota for ci in range(4)]
                uhalf = [urows_v[slot, u, pl.ds(h * 16, 16)]
                         for h in range(D // 16)]
                for hh in range(D // 16):
                    for s in range(16):
                        rot = (iota + s) & 15
                        urot = _rotate_lanes(uhalf[hh], rot)
                        diag = rot + 16 * hh if hh else rot
                        for ci in range(4):
                            vals = plsc.load_gather(irows_v.at[slot],
                                                    [ridx[ci], diag])
                            accs[ci] = accs[ci] + vals * urot
                for ci in range(4):
                    out_v[slot, pl.ds(rbase + ci * 16, 16)] = accs[ci]
                return carry

            lax.fori_loop(0, CU, user_body, 0, unroll=False)
            out_copy(c, slot).start()

        def wait_in(c, slot):
            for cp in in_copies(c, slot):
                cp.wait()

        issue(0, 0)

        def pair_body(k, carry):
            c0 = 2 * k
            issue(c0 + 1, 1)
            wait_in(c0, 0)
            compute(c0, 0)

            @pl.when(k < NCHUNK // 2 - 1)
            def _():
                issue(c0 + 2, 0)

            wait_in(c0 + 1, 1)
            compute(c0 + 1, 1)
            return carry

        lax.fori_loop(0, NCHUNK // 2, pair_body, 0, unroll=False)
        out_copy(NCHUNK - 2, 0).wait()
        out_copy(NCHUNK - 1, 1).wait()

    return sc_kernel


def kernel(user_code, item_code, user_emb, item_emb):
    B, L = item_code.shape
    V, D = user_emb.shape
    relayout = _build_relayout_kernel(V, D, n_workers=32)
    uflat, iflat = relayout(user_emb.T, item_emb.T)
    vp = uflat.shape[0] // D
    sck = _build_main_kernel(B, L, D, n_workers=32, users_per_chunk=32)
    out_flat = sck(user_code, item_code.reshape(-1),
                   uflat.reshape(vp, D), iflat.reshape(vp, D))
    return out_flat.reshape(B, L)


# speedup vs baseline: 3.8061x; 1.1118x over previous
import functools

import jax
import jax.numpy as jnp
from jax import lax
from jax.experimental import pallas as pl
from jax.experimental.pallas import tpu as pltpu
from jax.experimental.pallas import tpu_sc as plsc


def _rotate_lanes(vec, idx):
    dn = lax.GatherDimensionNumbers(
        offset_dims=(), collapsed_slice_dims=(0,), start_index_map=(0,))
    return lax.gather(vec, idx[:, None], dn, slice_sizes=(1,),
                      mode=lax.GatherScatterMode.PROMISE_IN_BOUNDS)


def _build_relayout_kernel(V, D, n_workers):
    KD = D // 8
    NKR = (V + 127) // 128
    VP = NKR * 128
    PERW = (NKR + n_workers - 1) // n_workers
    mesh = plsc.VectorSubcoreMesh(core_axis_name="c", subcore_axis_name="s")
    NC = mesh.num_cores

    @functools.partial(
        pl.kernel,
        out_type=[jax.ShapeDtypeStruct((VP * D,), jnp.float32),
                  jax.ShapeDtypeStruct((VP * D,), jnp.float32)],
        mesh=mesh,
        compiler_params=pltpu.CompilerParams(
            needs_layout_passes=False, use_tc_tiling_on_sc=True,
            disable_bounds_checks=True),
        scratch_types=[
            pltpu.VMEM((D, 128), jnp.float32),
            pltpu.VMEM((D, 128), jnp.float32),
            pltpu.VMEM((128 * D,), jnp.float32),
            pltpu.VMEM((128 * D,), jnp.float32),
            pltpu.SemaphoreType.DMA,
            pltpu.SemaphoreType.DMA,
            pltpu.SemaphoreType.DMA,
            pltpu.SemaphoreType.DMA,
        ],
    )
    def relayout(uembt_hbm, iembt_hbm, uout_hbm, iout_hbm,
                 stage0_v, stage1_v, rows0_v, rows1_v,
                 sem0, sem1, osem0, osem1):
        wid = lax.axis_index("s") * NC + lax.axis_index("c")
        iota = lax.iota(jnp.int32, 16)
        stages = [stage0_v, stage1_v]
        rows = [rows0_v, rows1_v]
        sems = [sem0, sem1]
        osems = [osem0, osem1]
        rots = [(iota + s) & 15 for s in range(16)]
        rots_hi = [r + 16 for r in rots]

        for src_hbm, out_hbm in ((uembt_hbm, uout_hbm), (iembt_hbm, iout_hbm)):
            def stage_copies(kc, slot, src_hbm=src_hbm):
                return [pltpu.make_async_copy(
                    src_hbm.at[:, pl.ds(kc * 128, 128)],
                    stages[slot], sems[slot])]

            def out_copy(kc, slot, out_hbm=out_hbm):
                return pltpu.make_async_copy(
                    rows[slot],
                    out_hbm.at[pl.ds(kc * 128 * D, 128 * D)], osems[slot])

            def transpose(kc, slot):
                @plsc.parallel_loop(0, 8, unroll=4)
                def rr_body(rr):
                    r16 = rr * 16 + iota
                    r16d = r16 * D
                    for s in range(16):
                        for dvec in (rots[s], rots_hi[s]):
                            vals = plsc.load_gather(stages[slot],
                                                    [dvec, r16])
                            plsc.store_scatter(rows[slot],
                                               [r16d + dvec], vals)

                out_copy(kc, slot).start()

            def issue(kc, slot):
                for cp in stage_copies(kc, slot):
                    cp.start()

            def wait_in(kc, slot):
                for cp in stage_copies(kc, slot):
                    cp.wait()

            nk_pairs = (PERW + 1) // 2
            clamp = lambda kc: jnp.minimum(kc, NKR - 1)
            issue(clamp(wid), 0)

            def pair_body(k, carry):
                kc0 = wid + 2 * k * n_workers
                issue(clamp(kc0 + n_workers), 1)
                wait_in(clamp(kc0), 0)

                @pl.when(k >= 1)
                def _():
                    out_copy(0, 0).wait()

                transpose(clamp(kc0), 0)

                @pl.when(k < nk_pairs - 1)
                def _():
                    issue(clamp(kc0 + 2 * n_workers), 0)

                wait_in(clamp(kc0 + n_workers), 1)

                @pl.when(k >= 1)
                def _():
                    out_copy(0, 1).wait()

                transpose(clamp(kc0 + n_workers), 1)
                return carry

            lax.fori_loop(0, nk_pairs, pair_body, 0, unroll=False)
            out_copy(0, 0).wait()
            out_copy(0, 1).wait()

    return relayout


def _build_main_kernel(B, L, D, n_workers, users_per_chunk):
    CU = users_per_chunk
    CI = CU * L
    BPW = B // n_workers
    NCHUNK = BPW // CU
    assert NCHUNK % 2 == 0
    groups = []
    off = 0
    while off < CI:
        sz = min(128, CI - off)
        groups.append((off, sz))
        off += sz

    mesh = plsc.VectorSubcoreMesh(core_axis_name="c", subcore_axis_name="s")
    NC = mesh.num_cores

    @functools.partial(
        pl.kernel,
        out_type=jax.ShapeDtypeStruct((B * L,), jnp.float32),
        mesh=mesh,
        compiler_params=pltpu.CompilerParams(
            needs_layout_passes=False, use_tc_tiling_on_sc=False),
        scratch_types=[
            pltpu.VMEM((2, CU), jnp.int32),
            pltpu.VMEM((2, CI), jnp.int32),
            pltpu.VMEM((2, CU, D), jnp.float32),
            pltpu.VMEM((2, CI + 16, D), jnp.float32),
            pltpu.VMEM((2, CI + 16), jnp.float32),
            pltpu.SemaphoreType.DMA,
            pltpu.SemaphoreType.DMA,
            pltpu.SemaphoreType.DMA,
            pltpu.SemaphoreType.DMA,
        ],
    )
    def sc_kernel(ucode_hbm, icode_hbm, uemb_hbm, iemb_hbm, out_hbm,
                  ucode_v, icode_v, urows_v, irows_v, out_v,
                  sem0, sem1, osem0, osem1):
        wid = lax.axis_index("s") * NC + lax.axis_index("c")
        iota = lax.iota(jnp.int32, 16)
        sems = [sem0, sem1]
        osems = [osem0, osem1]

        def in_copies(c, slot):
            cps = [pltpu.make_async_copy(
                uemb_hbm.at[ucode_v.at[slot]], urows_v.at[slot], sems[slot])]
            for goff, gsz in groups:
                cps.append(pltpu.make_async_copy(
                    iemb_hbm.at[icode_v.at[slot, pl.ds(goff, gsz)]],
                    irows_v.at[slot, pl.ds(goff, gsz)],
                    sems[slot]))
            return cps

        def out_copy(c, slot):
            ibase = (wid * BPW + c * CU) * L
            return pltpu.make_async_copy(
                out_v.at[slot, pl.ds(0, CI)],
                out_hbm.at[pl.ds(ibase, CI)], osems[slot])

        def issue(c, slot):
            ubase = wid * BPW + c * CU
            ibase = ubase * L
            pltpu.sync_copy(ucode_hbm.at[pl.ds(ubase, CU)],
                            ucode_v.at[slot])
            pltpu.sync_copy(icode_hbm.at[pl.ds(ibase, CI)],
                            icode_v.at[slot])
            for cp in in_copies(c, slot):
                cp.start()

        def compute(c, slot):
            @pl.when(c >= 2)
            def _():
                out_copy(c - 2, slot).wait()

            def user_body(u, carry):
                rbase = u * L
                accs = [jnp.zeros((16,), jnp.float32) for _ in range(4)]
                ridx = [rbase + ci * 16 + iota for ci in range(4)]
                uhalf = [urows_v[slot, u, pl.ds(h * 16, 16)]
                         for h in range(D // 16)]
                for hh in range(D // 16):
                    for s in range(16):
                        rot = (iota + s) & 15
                        urot = _rotate_lanes(uhalf[hh], rot)
                        diag = rot + 16 * hh if hh else rot
                        for ci in range(4):
                            vals = plsc.load_gather(irows_v.at[slot],
                                                    [ridx[ci], diag])
                            accs[ci] = accs[ci] + vals * urot
                for ci in range(4):
                    out_v[slot, pl.ds(rbase + ci * 16, 16)] = accs[ci]
                return carry

            lax.fori_loop(0, CU, user_body, 0, unroll=False)
            out_copy(c, slot).start()

        def wait_in(c, slot):
            for cp in in_copies(c, slot):
                cp.wait()

        issue(0, 0)

        def pair_body(k, carry):
            c0 = 2 * k
            issue(c0 + 1, 1)
            wait_in(c0, 0)
            compute(c0, 0)

            @pl.when(k < NCHUNK // 2 - 1)
            def _():
                issue(c0 + 2, 0)

            wait_in(c0 + 1, 1)
            compute(c0 + 1, 1)
            return carry

        lax.fori_loop(0, NCHUNK // 2, pair_body, 0, unroll=False)
        out_copy(NCHUNK - 2, 0).wait()
        out_copy(NCHUNK - 1, 1).wait()

    return sc_kernel


def kernel(user_code, item_code, user_emb, item_emb):
    B, L = item_code.shape
    V, D = user_emb.shape
    relayout = _build_relayout_kernel(V, D, n_workers=32)
    uflat, iflat = relayout(user_emb.T, item_emb.T)
    vp = uflat.shape[0] // D
    sck = _build_main_kernel(B, L, D, n_workers=32, users_per_chunk=32)
    out_flat = sck(user_code, item_code.reshape(-1),
                   uflat.reshape(vp, D), iflat.reshape(vp, D))
    return out_flat.reshape(B, L)


# speedup vs baseline: 3.8078x; 1.0005x over previous
import functools

import jax
import jax.numpy as jnp
from jax import lax
from jax.experimental import pallas as pl
from jax.experimental.pallas import tpu as pltpu
from jax.experimental.pallas import tpu_sc as plsc


def _rotate_lanes(vec, idx):
    dn = lax.GatherDimensionNumbers(
        offset_dims=(), collapsed_slice_dims=(0,), start_index_map=(0,))
    return lax.gather(vec, idx[:, None], dn, slice_sizes=(1,),
                      mode=lax.GatherScatterMode.PROMISE_IN_BOUNDS)


def _build_relayout_kernel(V, D, n_workers):
    KD = D // 8
    NKR = (V + 127) // 128
    VP = NKR * 128
    PERW = (NKR + n_workers - 1) // n_workers
    mesh = plsc.VectorSubcoreMesh(core_axis_name="c", subcore_axis_name="s")
    NC = mesh.num_cores

    @functools.partial(
        pl.kernel,
        out_type=[jax.ShapeDtypeStruct((VP * D,), jnp.float32),
                  jax.ShapeDtypeStruct((VP * D,), jnp.float32)],
        mesh=mesh,
        compiler_params=pltpu.CompilerParams(
            needs_layout_passes=False, use_tc_tiling_on_sc=True,
            disable_bounds_checks=True),
        scratch_types=[
            pltpu.VMEM((D, 128), jnp.float32),
            pltpu.VMEM((D, 128), jnp.float32),
            pltpu.VMEM((128 * D,), jnp.float32),
            pltpu.VMEM((128 * D,), jnp.float32),
            pltpu.SemaphoreType.DMA,
            pltpu.SemaphoreType.DMA,
            pltpu.SemaphoreType.DMA,
            pltpu.SemaphoreType.DMA,
        ],
    )
    def relayout(uembt_hbm, iembt_hbm, uout_hbm, iout_hbm,
                 stage0_v, stage1_v, rows0_v, rows1_v,
                 sem0, sem1, osem0, osem1):
        wid = lax.axis_index("s") * NC + lax.axis_index("c")
        iota = lax.iota(jnp.int32, 16)
        stages = [stage0_v, stage1_v]
        rows = [rows0_v, rows1_v]
        sems = [sem0, sem1]
        osems = [osem0, osem1]
        rots = [(iota + s) & 15 for s in range(16)]

        for src_hbm, out_hbm in ((uembt_hbm, uout_hbm), (iembt_hbm, iout_hbm)):
            def stage_copies(kc, slot, src_hbm=src_hbm):
                return [pltpu.make_async_copy(
                    src_hbm.at[:, pl.ds(kc * 128, 128)],
                    stages[slot], sems[slot])]

            def out_copy(kc, slot, out_hbm=out_hbm):
                return pltpu.make_async_copy(
                    rows[slot],
                    out_hbm.at[pl.ds(kc * 128 * D, 128 * D)], osems[slot])

            def transpose(kc, slot):
                @plsc.parallel_loop(0, 8, unroll=4)
                def rr_body(rr):
                    r16 = rr * 16 + iota
                    r16d = r16 * D
                    for s in range(16):
                        for dvec in (rots[s], rots[s] + 16):
                            vals = plsc.load_gather(stages[slot],
                                                    [dvec, r16])
                            plsc.store_scatter(rows[slot],
                                               [r16d + dvec], vals)

                out_copy(kc, slot).start()

            def issue(kc, slot):
                for cp in stage_copies(kc, slot):
                    cp.start()

            def wait_in(kc, slot):
                for cp in stage_copies(kc, slot):
                    cp.wait()

            nk_pairs = (PERW + 1) // 2
            clamp = lambda kc: jnp.minimum(kc, NKR - 1)
            issue(clamp(wid), 0)

            def pair_body(k, carry):
                kc0 = wid + 2 * k * n_workers
                issue(clamp(kc0 + n_workers), 1)
                wait_in(clamp(kc0), 0)

                @pl.when(k >= 1)
                def _():
                    out_copy(0, 0).wait()

                transpose(clamp(kc0), 0)

                @pl.when(k < nk_pairs - 1)
                def _():
                    issue(clamp(kc0 + 2 * n_workers), 0)

                wait_in(clamp(kc0 + n_workers), 1)

                @pl.when(k >= 1)
                def _():
                    out_copy(0, 1).wait()

                transpose(clamp(kc0 + n_workers), 1)
                return carry

            lax.fori_loop(0, nk_pairs, pair_body, 0, unroll=False)
            out_copy(0, 0).wait()
            out_copy(0, 1).wait()

    return relayout


def _build_main_kernel(B, L, D, n_workers, users_per_chunk):
    CU = users_per_chunk
    CI = CU * L
    BPW = B // n_workers
    NCHUNK = BPW // CU
    assert NCHUNK % 2 == 0
    groups = []
    off = 0
    while off < CI:
        sz = min(128, CI - off)
        groups.append((off, sz))
        off += sz

    mesh = plsc.VectorSubcoreMesh(core_axis_name="c", subcore_axis_name="s")
    NC = mesh.num_cores

    @functools.partial(
        pl.kernel,
        out_type=jax.ShapeDtypeStruct((B * L,), jnp.float32),
        mesh=mesh,
        compiler_params=pltpu.CompilerParams(
            needs_layout_passes=False, use_tc_tiling_on_sc=False),
        scratch_types=[
            pltpu.VMEM((2, CU), jnp.int32),
            pltpu.VMEM((2, CI), jnp.int32),
            pltpu.VMEM((2, CU, D), jnp.float32),
            pltpu.VMEM((2, CI + 16, D), jnp.float32),
            pltpu.VMEM((2, CI + 16), jnp.float32),
            pltpu.SemaphoreType.DMA,
            pltpu.SemaphoreType.DMA,
            pltpu.SemaphoreType.DMA,
            pltpu.SemaphoreType.DMA,
        ],
    )
    def sc_kernel(ucode_hbm, icode_hbm, uemb_hbm, iemb_hbm, out_hbm,
                  ucode_v, icode_v, urows_v, irows_v, out_v,
                  sem0, sem1, osem0, osem1):
        wid = lax.axis_index("s") * NC + lax.axis_index("c")
        iota = lax.iota(jnp.int32, 16)
        sems = [sem0, sem1]
        osems = [osem0, osem1]

        def in_copies(c, slot):
            cps = [pltpu.make_async_copy(
                uemb_hbm.at[ucode_v.at[slot]], urows_v.at[slot], sems[slot])]
            for goff, gsz in groups:
                cps.append(pltpu.make_async_copy(
                    iemb_hbm.at[icode_v.at[slot, pl.ds(goff, gsz)]],
                    irows_v.at[slot, pl.ds(goff, gsz)],
                    sems[slot]))
            return cps

        def out_copy(c, slot):
            ibase = (wid * BPW + c * CU) * L
            return pltpu.make_async_copy(
                out_v.at[slot, pl.ds(0, CI)],
                out_hbm.at[pl.ds(ibase, CI)], osems[slot])

        def issue(c, slot):
            ubase = wid * BPW + c * CU
            ibase = ubase * L
            pltpu.sync_copy(ucode_hbm.at[pl.ds(ubase, CU)],
                            ucode_v.at[slot])
            pltpu.sync_copy(icode_hbm.at[pl.ds(ibase, CI)],
                            icode_v.at[slot])
            for cp in in_copies(c, slot):
                cp.start()

        def compute(c, slot):
            @pl.when(c >= 2)
            def _():
                out_copy(c - 2, slot).wait()

            def user_body(u, carry):
                rbase = u * L
                accs = [jnp.zeros((16,), jnp.float32) for _ in range(4)]
                ridx = [rbase + ci * 16 + iota for ci in range(4)]
                uhalf = [urows_v[slot, u, pl.ds(h * 16, 16)]
                         for h in range(D // 16)]
                for hh in range(D // 16):
                    for s in range(16):
                        rot = (iota + s) & 15
                        urot = _rotate_lanes(uhalf[hh], rot)
                        diag = rot + 16 * hh if hh else rot
                        for ci in range(4):
                            vals = plsc.load_gather(irows_v.at[slot],
                                                    [ridx[ci], diag])
                            accs[ci] = accs[ci] + vals * urot
                for ci in range(4):
                    out_v[slot, pl.ds(rbase + ci * 16, 16)] = accs[ci]
                return carry

            lax.fori_loop(0, CU, user_body, 0, unroll=False)
            out_copy(c, slot).start()

        def wait_in(c, slot):
            for cp in in_copies(c, slot):
                cp.wait()

        issue(0, 0)

        def pair_body(k, carry):
            c0 = 2 * k
            issue(c0 + 1, 1)
            wait_in(c0, 0)
            compute(c0, 0)

            @pl.when(k < NCHUNK // 2 - 1)
            def _():
                issue(c0 + 2, 0)

            wait_in(c0 + 1, 1)
            compute(c0 + 1, 1)
            return carry

        lax.fori_loop(0, NCHUNK // 2, pair_body, 0, unroll=False)
        out_copy(NCHUNK - 2, 0).wait()
        out_copy(NCHUNK - 1, 1).wait()

    return sc_kernel


def kernel(user_code, item_code, user_emb, item_emb):
    B, L = item_code.shape
    V, D = user_emb.shape
    relayout = _build_relayout_kernel(V, D, n_workers=32)
    uflat, iflat = relayout(user_emb.T, item_emb.T)
    vp = uflat.shape[0] // D
    sck = _build_main_kernel(B, L, D, n_workers=32, users_per_chunk=32)
    out_flat = sck(user_code, item_code.reshape(-1),
                   uflat.reshape(vp, D), iflat.reshape(vp, D))
    return out_flat.reshape(B, L)
